# baseline (speedup 1.0000x reference)
import functools

import jax
import jax.numpy as jnp
from jax import lax
from jax.experimental import pallas as pl
from jax.experimental.pallas import tpu as pltpu
from jax.experimental.pallas import tpu_sc as plsc

_H = 8
_C = 16


def _node_projections(x, W_l, b_l, W_r, b_r):
    n, d = x.shape
    f = W_l.shape[1]
    bn = 1000
    body = lambda x_ref, wl, bl, wr, br, xl_ref, xr_ref: (
        xl_ref.__setitem__(
            ..., jnp.dot(x_ref[...], wl[...], preferred_element_type=jnp.float32) + bl[...]),
        xr_ref.__setitem__(
            ..., jnp.dot(x_ref[...], wr[...], preferred_element_type=jnp.float32) + br[...]),
    ) and None
    return pl.pallas_call(
        body,
        grid=(n // bn,),
        in_specs=[
            pl.BlockSpec((bn, d), lambda i: (i, 0)),
            pl.BlockSpec((d, f), lambda i: (0, 0)),
            pl.BlockSpec((1, f), lambda i: (0, 0)),
            pl.BlockSpec((d, f), lambda i: (0, 0)),
            pl.BlockSpec((1, f), lambda i: (0, 0)),
        ],
        out_specs=[pl.BlockSpec((bn, f), lambda i: (i, 0))] * 2,
        out_shape=[jax.ShapeDtypeStruct((n, f), jnp.float32)] * 2,
    )(x, W_l, b_l.reshape(1, -1), W_r, b_r.reshape(1, -1))


def _edge_projection(edge_attr, W_e):
    e, de = edge_attr.shape
    f = W_e.shape[1]
    be = 4000
    body = lambda a_ref, w_ref, o_ref: o_ref.__setitem__(
        ..., jnp.dot(a_ref[...], w_ref[...], preferred_element_type=jnp.float32))
    return pl.pallas_call(
        body,
        grid=(e // be,),
        in_specs=[
            pl.BlockSpec((be, de), lambda i: (i, 0)),
            pl.BlockSpec((de, f), lambda i: (0, 0)),
        ],
        out_specs=pl.BlockSpec((be, f), lambda i: (i, 0)),
        out_shape=jax.ShapeDtypeStruct((e, f), jnp.float32),
    )(edge_attr, W_e)


def _sc_message_pass(xl, xr, e, src, dst, att):
    n, f = xl.shape
    num_edges = src.shape[0]
    fd = f + _C
    nc, ns = 2, 16
    tiles = nc * ns
    ept = num_edges // tiles
    eb = 40
    nb = ept // eb
    ch = 128
    rpt = -(-n // (ns * ch)) * ch
    npad = ns * rpt
    nch = rpt // ch

    sbsz = 2000
    nsb = ept // sbsz
    nbs = sbsz // eb

    def body(xl_h, xr_h, e_h, src_h, dst_h, att_h, part_h,
             acc, sall, dall, didx0, didx1, xl0, xl1, xr0, xr1,
             ebuf, cb, attv, semg, seme, semi):
        cid = lax.axis_index("c")
        sid = lax.axis_index("s")
        wid = sid * nc + cid
        zv = jnp.zeros((16,), jnp.float32)
        didxs, xlsl, xrsl = (didx0, didx1), (xl0, xl1), (xr0, xr1)

        def zrow(i, carry):
            for j in range(fd // 16):
                cb[i, pl.ds(j * 16, 16)] = zv
            return carry
        lax.fori_loop(0, eb, zrow, 0)
        row0 = sid * rpt
        for k in range(rpt // eb):
            pltpu.sync_copy(cb, acc.at[pl.ds(row0 + k * eb, eb)])
        pltpu.sync_copy(att_h, attv)
        plsc.subcore_barrier()

        atts = [attv[h, :] for h in range(_H)]
        ohs = [jnp.where(lax.iota(jnp.int32, 16) == h, 1.0, 0.0)
               for h in range(_H)]
        ebase = wid * ept

        def issue(sbase, g, slot):
            off = g * eb
            pltpu.async_copy(dst_h.at[pl.ds(sbase + off, eb)],
                             didxs[slot], semi)
            pltpu.async_copy(xl_h.at[sall.at[pl.ds(off, eb)]],
                             xlsl[slot], semg)
            pltpu.async_copy(xr_h.at[dall.at[pl.ds(off, eb)]],
                             xrsl[slot], semg)

        def wait_in(slot):
            pltpu.make_async_copy(xl_h.at[didxs[slot]], xlsl[slot], semg).wait()
            pltpu.make_async_copy(xl_h.at[didxs[slot]], xrsl[slot], semg).wait()
            pltpu.make_async_copy(dst_h.at[pl.ds(0, eb)], didxs[slot],
                                  semi).wait()
            pltpu.make_async_copy(e_h.at[pl.ds(0, eb)], ebuf, seme).wait()

        def compute(slot):
            xlb, xrb = xlsl[slot], xrsl[slot]

            @plsc.parallel_loop(0, eb, unroll=6)
            def edge(i):
                dvec = zv
                for h in range(_H):
                    sl = pl.ds(h * 16, 16)
                    xlv = xlb[i, sl]
                    m = xlv + xrb[i, sl] + ebuf[i, sl]
                    m = jnp.maximum(m, 0.2 * m)
                    logit = jnp.sum(m * atts[h])
                    wv = jnp.exp(lax.broadcast(logit, (16,)))
                    cb[i, sl] = wv * xlv
                    dvec = dvec + wv * ohs[h]
                cb[i, pl.ds(f, 16)] = dvec

        def superbatch(s, carry):
            sbase = ebase + s * sbsz
            pltpu.sync_copy(src_h.at[pl.ds(sbase, sbsz)], sall)
            pltpu.sync_copy(dst_h.at[pl.ds(sbase, sbsz)], dall)
            issue(sbase, 0, 0)
            pltpu.async_copy(e_h.at[pl.ds(sbase, eb)], ebuf, seme)

            def pair(p, c2):
                for sl in range(2):
                    g = p * 2 + sl
                    nxt = 1 - sl

                    @pl.when(g + 1 < nbs)
                    def _():
                        issue(sbase, g + 1, nxt)

                    wait_in(sl)
                    compute(sl)

                    @pl.when(g + 1 < nbs)
                    def _():
                        pltpu.async_copy(
                            e_h.at[pl.ds(sbase + (g + 1) * eb, eb)],
                            ebuf, seme)

                    pltpu.sync_copy(cb, acc.at[didxs[sl]], add=True)
                return c2
            lax.fori_loop(0, nbs // 2, pair, 0)
            return carry
        lax.fori_loop(0, nsb, superbatch, 0)
        plsc.subcore_barrier()

        for k in range(nch):
            r = row0 + k * ch
            pltpu.sync_copy(acc.at[pl.ds(r, ch)], part_h.at[cid, pl.ds(r, ch)])

    mesh = plsc.VectorSubcoreMesh(core_axis_name="c", subcore_axis_name="s")
    kfn = pl.kernel(
        body,
        out_type=jax.ShapeDtypeStruct((nc, npad, fd), jnp.float32),
        mesh=mesh,
        compiler_params=pltpu.CompilerParams(
            needs_layout_passes=False, use_tc_tiling_on_sc=False),
        scratch_types=[
            pltpu.VMEM_SHARED((npad, fd), jnp.float32),
            pltpu.VMEM((sbsz,), jnp.int32),
            pltpu.VMEM((sbsz,), jnp.int32),
            pltpu.VMEM((eb,), jnp.int32),
            pltpu.VMEM((eb,), jnp.int32),
            pltpu.VMEM((eb, f), jnp.float32),
            pltpu.VMEM((eb, f), jnp.float32),
            pltpu.VMEM((eb, f), jnp.float32),
            pltpu.VMEM((eb, f), jnp.float32),
            pltpu.VMEM((eb, f), jnp.float32),
            pltpu.VMEM((eb, fd), jnp.float32),
            pltpu.VMEM((_H, _C), jnp.float32),
            pltpu.SemaphoreType.DMA,
            pltpu.SemaphoreType.DMA,
            pltpu.SemaphoreType.DMA,
        ],
    )
    return kfn(xl, xr, e, src, dst, att)


def _finalize(part, x, W_res, b_res, bias, gamma, beta):
    nc, _, fd = part.shape
    f = fd - _C
    n, d = x.shape
    bn = 1000

    def body(p0, p1, x_ref, wres, bres, bias2, gamma2, beta2, out_ref):
        p = p0[0] + p1[0]
        num = p[:, :f]
        den = p[:, f:fd]
        r = lax.broadcasted_iota(jnp.int32, (_C, f), 0)
        c = lax.broadcasted_iota(jnp.int32, (_C, f), 1)
        expand = (c // _C == r).astype(jnp.float32)
        den_b = jnp.dot(den, expand, preferred_element_type=jnp.float32)
        o = (num / (den_b + 1e-16) + bias2[...] + bres[...]
             + jnp.dot(x_ref[...], wres[...], preferred_element_type=jnp.float32))
        mu = jnp.mean(o, axis=-1, keepdims=True)
        dev = o - mu
        var = jnp.mean(dev * dev, axis=-1, keepdims=True)
        out_ref[...] = dev * lax.rsqrt(var + 1e-5) * gamma2[...] + beta2[...]

    return pl.pallas_call(
        body,
        grid=(n // bn,),
        in_specs=[
            pl.BlockSpec((1, bn, fd), lambda i: (0, i, 0)),
            pl.BlockSpec((1, bn, fd), lambda i: (1, i, 0)),
            pl.BlockSpec((bn, d), lambda i: (i, 0)),
            pl.BlockSpec((d, f), lambda i: (0, 0)),
            pl.BlockSpec((1, f), lambda i: (0, 0)),
            pl.BlockSpec((1, f), lambda i: (0, 0)),
            pl.BlockSpec((1, f), lambda i: (0, 0)),
            pl.BlockSpec((1, f), lambda i: (0, 0)),
        ],
        out_specs=pl.BlockSpec((bn, f), lambda i: (i, 0)),
        out_shape=jax.ShapeDtypeStruct((n, f), jnp.float32),
    )(part, part, x, W_res, b_res.reshape(1, -1), bias.reshape(1, -1),
      gamma.reshape(1, -1), beta.reshape(1, -1))


def kernel(x, edge_index, edge_attr, W_l, b_l, W_r, b_r, W_e, att,
           W_res, b_res, bias, gamma, beta):
    src = edge_index[0].astype(jnp.int32)
    dst = edge_index[1].astype(jnp.int32)
    xl, xr = _node_projections(x, W_l, b_l, W_r, b_r)
    e = _edge_projection(edge_attr, W_e)
    part = _sc_message_pass(xl, xr, e, src, dst, att)
    return _finalize(part, x, W_res, b_res, bias, gamma, beta)


# speedup vs baseline: 1.3128x; 1.3128x over previous
import functools

import jax
import jax.numpy as jnp
from jax import lax
from jax.experimental import pallas as pl
from jax.experimental.pallas import tpu as pltpu
from jax.experimental.pallas import tpu_sc as plsc

_H = 8
_C = 16


def _node_projections(x, W_l, b_l, W_r, b_r):
    n, d = x.shape
    f = W_l.shape[1]
    bn = 1000
    body = lambda x_ref, wl, bl, wr, br, xl_ref, xr_ref: (
        xl_ref.__setitem__(
            ..., jnp.dot(x_ref[...], wl[...], preferred_element_type=jnp.float32) + bl[...]),
        xr_ref.__setitem__(
            ..., jnp.dot(x_ref[...], wr[...], preferred_element_type=jnp.float32) + br[...]),
    ) and None
    return pl.pallas_call(
        body,
        grid=(n // bn,),
        in_specs=[
            pl.BlockSpec((bn, d), lambda i: (i, 0)),
            pl.BlockSpec((d, f), lambda i: (0, 0)),
            pl.BlockSpec((1, f), lambda i: (0, 0)),
            pl.BlockSpec((d, f), lambda i: (0, 0)),
            pl.BlockSpec((1, f), lambda i: (0, 0)),
        ],
        out_specs=[pl.BlockSpec((bn, f), lambda i: (i, 0))] * 2,
        out_shape=[jax.ShapeDtypeStruct((n, f), jnp.float32)] * 2,
    )(x, W_l, b_l.reshape(1, -1), W_r, b_r.reshape(1, -1))


def _edge_projection(edge_attr, W_e):
    e, de = edge_attr.shape
    f = W_e.shape[1]
    be = 4000
    body = lambda a_ref, w_ref, o_ref: o_ref.__setitem__(
        ..., jnp.dot(a_ref[...], w_ref[...], preferred_element_type=jnp.float32))
    return pl.pallas_call(
        body,
        grid=(e // be,),
        in_specs=[
            pl.BlockSpec((be, de), lambda i: (i, 0)),
            pl.BlockSpec((de, f), lambda i: (0, 0)),
        ],
        out_specs=pl.BlockSpec((be, f), lambda i: (i, 0)),
        out_shape=jax.ShapeDtypeStruct((e, f), jnp.float32),
    )(edge_attr, W_e)


def _sc_message_pass(xl, xr, e, src, dst, att):
    n, f = xl.shape
    num_edges = src.shape[0]
    fd = f + _C
    nc, ns = 2, 16
    tiles = nc * ns
    ept = num_edges // tiles
    eb = 40
    nb = ept // eb
    ch = 128
    rpt = -(-n // (ns * ch)) * ch
    npad = ns * rpt
    nch = rpt // ch

    sbsz = 2000
    nsb = ept // sbsz
    nbs = sbsz // eb

    def body(xl_h, xr_h, e_h, src_h, dst_h, att_h, part_h,
             acc, sall, dall, didx0, didx1, xl0, xl1, xr0, xr1,
             ebuf, cb, attv, semg, seme, semi):
        cid = lax.axis_index("c")
        sid = lax.axis_index("s")
        wid = sid * nc + cid
        zv = jnp.zeros((16,), jnp.float32)
        didxs, xlsl, xrsl = (didx0, didx1), (xl0, xl1), (xr0, xr1)

        def zrow(i, carry):
            for j in range(fd // 16):
                cb[i, pl.ds(j * 16, 16)] = zv
            return carry
        lax.fori_loop(0, eb, zrow, 0)
        row0 = sid * rpt
        for k in range(rpt // eb):
            pltpu.sync_copy(cb, acc.at[pl.ds(row0 + k * eb, eb)])
        pltpu.sync_copy(att_h, attv)
        plsc.subcore_barrier()

        atts = [attv[h, :] for h in range(_H)]
        ohs = [jnp.where(lax.iota(jnp.int32, 16) == h, 1.0, 0.0)
               for h in range(_H)]
        ebase = wid * ept

        def issue(sbase, g, slot):
            off = g * eb
            pltpu.async_copy(dst_h.at[pl.ds(sbase + off, eb)],
                             didxs[slot], semi)
            pltpu.async_copy(xl_h.at[sall.at[pl.ds(off, eb)]],
                             xlsl[slot], semg)
            pltpu.async_copy(xr_h.at[dall.at[pl.ds(off, eb)]],
                             xrsl[slot], semg)

        def wait_in(slot):
            pltpu.make_async_copy(xl_h.at[didxs[slot]], xlsl[slot], semg).wait()
            pltpu.make_async_copy(xl_h.at[didxs[slot]], xrsl[slot], semg).wait()
            pltpu.make_async_copy(dst_h.at[pl.ds(0, eb)], didxs[slot],
                                  semi).wait()
            pltpu.make_async_copy(e_h.at[pl.ds(0, eb)], ebuf, seme).wait()

        def compute(slot):
            xlb, xrb = xlsl[slot], xrsl[slot]

            @plsc.parallel_loop(0, eb, unroll=5)
            def edge(i):
                dvec = zv
                for h in range(_H):
                    sl = pl.ds(h * 16, 16)
                    xlv = xlb[i, sl]
                    m = xlv + xrb[i, sl] + ebuf[i, sl]
                    m = jnp.maximum(m, 0.2 * m)
                    logit = jnp.sum(m * atts[h])
                    wv = jnp.exp(lax.broadcast(logit, (16,)))
                    cb[i, sl] = wv * xlv
                    dvec = dvec + wv * ohs[h]
                cb[i, pl.ds(f, 16)] = dvec

        def superbatch(s, carry):
            sbase = ebase + s * sbsz
            pltpu.sync_copy(src_h.at[pl.ds(sbase, sbsz)], sall)
            pltpu.sync_copy(dst_h.at[pl.ds(sbase, sbsz)], dall)
            issue(sbase, 0, 0)
            pltpu.async_copy(e_h.at[pl.ds(sbase, eb)], ebuf, seme)

            def pair(p, c2):
                for sl in range(2):
                    g = p * 2 + sl
                    nxt = 1 - sl

                    @pl.when(g + 1 < nbs)
                    def _():
                        issue(sbase, g + 1, nxt)

                    wait_in(sl)
                    compute(sl)

                    @pl.when(g + 1 < nbs)
                    def _():
                        pltpu.async_copy(
                            e_h.at[pl.ds(sbase + (g + 1) * eb, eb)],
                            ebuf, seme)

                    pltpu.sync_copy(cb, acc.at[didxs[sl]], add=True)
                return c2
            lax.fori_loop(0, nbs // 2, pair, 0)
            return carry
        lax.fori_loop(0, nsb, superbatch, 0)
        plsc.subcore_barrier()

        for k in range(nch):
            r = row0 + k * ch
            pltpu.sync_copy(acc.at[pl.ds(r, ch)], part_h.at[cid, pl.ds(r, ch)])

    mesh = plsc.VectorSubcoreMesh(core_axis_name="c", subcore_axis_name="s")
    kfn = pl.kernel(
        body,
        out_type=jax.ShapeDtypeStruct((nc, npad, fd), jnp.float32),
        mesh=mesh,
        compiler_params=pltpu.CompilerParams(
            needs_layout_passes=False, use_tc_tiling_on_sc=False),
        scratch_types=[
            pltpu.VMEM_SHARED((npad, fd), jnp.float32),
            pltpu.VMEM((sbsz,), jnp.int32),
            pltpu.VMEM((sbsz,), jnp.int32),
            pltpu.VMEM((eb,), jnp.int32),
            pltpu.VMEM((eb,), jnp.int32),
            pltpu.VMEM((eb, f), jnp.float32),
            pltpu.VMEM((eb, f), jnp.float32),
            pltpu.VMEM((eb, f), jnp.float32),
            pltpu.VMEM((eb, f), jnp.float32),
            pltpu.VMEM((eb, f), jnp.float32),
            pltpu.VMEM((eb, fd), jnp.float32),
            pltpu.VMEM((_H, _C), jnp.float32),
            pltpu.SemaphoreType.DMA,
            pltpu.SemaphoreType.DMA,
            pltpu.SemaphoreType.DMA,
        ],
    )
    return kfn(xl, xr, e, src, dst, att)


def _finalize(part, x, W_res, b_res, bias, gamma, beta):
    nc, _, fd = part.shape
    f = fd - _C
    n, d = x.shape
    bn = 1000

    def body(p0, p1, x_ref, wres, bres, bias2, gamma2, beta2, out_ref):
        p = p0[0] + p1[0]
        num = p[:, :f]
        den = p[:, f:fd]
        r = lax.broadcasted_iota(jnp.int32, (_C, f), 0)
        c = lax.broadcasted_iota(jnp.int32, (_C, f), 1)
        expand = (c // _C == r).astype(jnp.float32)
        den_b = jnp.dot(den, expand, preferred_element_type=jnp.float32)
        o = (num / (den_b + 1e-16) + bias2[...] + bres[...]
             + jnp.dot(x_ref[...], wres[...], preferred_element_type=jnp.float32))
        mu = jnp.mean(o, axis=-1, keepdims=True)
        dev = o - mu
        var = jnp.mean(dev * dev, axis=-1, keepdims=True)
        out_ref[...] = dev * lax.rsqrt(var + 1e-5) * gamma2[...] + beta2[...]

    return pl.pallas_call(
        body,
        grid=(n // bn,),
        in_specs=[
            pl.BlockSpec((1, bn, fd), lambda i: (0, i, 0)),
            pl.BlockSpec((1, bn, fd), lambda i: (1, i, 0)),
            pl.BlockSpec((bn, d), lambda i: (i, 0)),
            pl.BlockSpec((d, f), lambda i: (0, 0)),
            pl.BlockSpec((1, f), lambda i: (0, 0)),
            pl.BlockSpec((1, f), lambda i: (0, 0)),
            pl.BlockSpec((1, f), lambda i: (0, 0)),
            pl.BlockSpec((1, f), lambda i: (0, 0)),
        ],
        out_specs=pl.BlockSpec((bn, f), lambda i: (i, 0)),
        out_shape=jax.ShapeDtypeStruct((n, f), jnp.float32),
    )(part, part, x, W_res, b_res.reshape(1, -1), bias.reshape(1, -1),
      gamma.reshape(1, -1), beta.reshape(1, -1))


def kernel(x, edge_index, edge_attr, W_l, b_l, W_r, b_r, W_e, att,
           W_res, b_res, bias, gamma, beta):
    src = edge_index[0].astype(jnp.int32)
    dst = edge_index[1].astype(jnp.int32)
    xl, xr = _node_projections(x, W_l, b_l, W_r, b_r)
    e = _edge_projection(edge_attr, W_e)
    part = _sc_message_pass(xl, xr, e, src, dst, att)
    return _finalize(part, x, W_res, b_res, bias, gamma, beta)
